# Initial kernel scaffold; baseline (speedup 1.0000x reference)
import jax
import jax.numpy as jnp
from jax.experimental import pallas as pl


def kernel(x, rbf, pij, dij, idx_i, idx_j, params):
    raise NotImplementedError("write your pallas kernel here")


# speedup vs baseline: 8.5723x; 8.5723x over previous
import functools

import jax
import jax.numpy as jnp
from jax import lax
from jax.experimental import pallas as pl
from jax.experimental.pallas import tpu as pltpu
from jax.experimental.pallas import tpu_sc as plsc

N = 10000
P = 320000
F = 128
NBF = 32

NC = 2
NS = 16
NW = NC * NS
EPT = P // NW
BE = 400
NBLK = EPT // BE
SUB = 100
NSUB = BE // SUB
NCH = 9
FC = 16
NFC = F // FC
NPAD = 10240
RPT = NPAD // NS
ZROWS = 64
PCH = 5


def _swish(v):
    return v * jax.nn.sigmoid(v)


def _resmlp(v, w1t, b1, w2t, b2, wot, bot):
    y = _swish(v)
    y = jnp.dot(y, w1t, preferred_element_type=jnp.float32) + b1
    y = _swish(y)
    y = jnp.dot(y, w2t, preferred_element_type=jnp.float32) + b2
    v = v + y
    v = _swish(v)
    return jnp.dot(v, wot, preferred_element_type=jnp.float32) + bot


def _node_mlps_body(x_ref, *refs):
    w = [r[...] for r in refs[:24]]
    outs = refs[24:]
    xb = x_ref[...]
    for i in range(4):
        outs[i][...] = _resmlp(xb, *w[6 * i:6 * i + 6])


def _node_mlps(x, wflat):
    blk = 400
    grid = (N // blk,)
    wspecs = []
    for a in wflat:
        wspecs.append(pl.BlockSpec(a.shape, lambda i: (0,) * a.ndim))
    return pl.pallas_call(
        _node_mlps_body,
        grid=grid,
        in_specs=[pl.BlockSpec((blk, F), lambda i: (i, 0))] + wspecs,
        out_specs=[pl.BlockSpec((blk, F), lambda i: (i, 0))] * 4,
        out_shape=[jax.ShapeDtypeStruct((N, F), jnp.float32)] * 4,
    )(x, *wflat)


def _radial_body(rbf_ref, ws_ref, wp_ref, wd_ref, gs_ref, gp_ref, gd_ref):
    r = rbf_ref[...]
    gs_ref[...] = jnp.dot(r, ws_ref[...], preferred_element_type=jnp.float32)
    gp_ref[...] = jnp.dot(r, wp_ref[...], preferred_element_type=jnp.float32)
    gd_ref[...] = jnp.dot(r, wd_ref[...], preferred_element_type=jnp.float32)


def _radial(rbf, wst, wpt, wdt):
    blk = 3200
    grid = (P // blk,)
    wspec = pl.BlockSpec((NBF, F), lambda i: (0, 0))
    return pl.pallas_call(
        _radial_body,
        grid=grid,
        in_specs=[pl.BlockSpec((blk, NBF), lambda i: (i, 0)), wspec, wspec, wspec],
        out_specs=[pl.BlockSpec((blk, F), lambda i: (i, 0))] * 3,
        out_shape=[jax.ShapeDtypeStruct((P, F), jnp.float32)] * 3,
    )(rbf, wst, wpt, wdt)


def _sc_body(gs, gp, gd, xst, xpt, xdt, pijT, dijT, idxi_r, idxj8, out,
             idxi_v, idxa_v, g1_v, g2_v, x1_v, x2_v,
             w_v, msg_v, zbuf, acc):
    cid = lax.axis_index("c")
    sid = lax.axis_index("s")
    wid = cid * NS + sid
    erow0 = wid * (EPT // SUB)
    nb = sid * RPT
    zv = jnp.zeros((FC,), jnp.float32)

    def zb_body(i, _):
        for ch in range(PCH):
            zbuf[i, ch] = zv
        return 0

    lax.fori_loop(0, ZROWS, zb_body, 0)

    for c in range(NFC):
        for ps in range(2):
            for z in range(RPT // ZROWS):
                pltpu.sync_copy(zbuf, acc.at[pl.ds(nb + z * ZROWS, ZROWS)])
            plsc.subcore_barrier()

            def blk_body(blk, _):
                r0 = erow0 + blk * NSUB
                e0 = wid * EPT + blk * BE
                pltpu.sync_copy(idxi_r.at[pl.ds(r0, NSUB)], idxi_v)
                pltpu.sync_copy(idxj8.at[c, pl.ds(r0, NSUB)], idxa_v)
                if ps == 0:
                    pltpu.sync_copy(gs.at[pl.ds(e0, BE), pl.ds(c * FC, FC)],
                                    g1_v)
                    pltpu.sync_copy(gp.at[pl.ds(e0, BE), pl.ds(c * FC, FC)],
                                    g2_v)
                    pltpu.sync_copy(pijT.at[:, pl.ds(e0, BE)],
                                    w_v.at[pl.ds(0, 3)])
                    for k in range(NSUB):
                        pltpu.sync_copy(xst.at[idxa_v.at[k]],
                                        x1_v.at[pl.ds(k * SUB, SUB)])
                        pltpu.sync_copy(xpt.at[idxa_v.at[k]],
                                        x2_v.at[pl.ds(k * SUB, SUB)])
                else:
                    pltpu.sync_copy(gd.at[pl.ds(e0, BE), pl.ds(c * FC, FC)],
                                    g1_v)
                    pltpu.sync_copy(dijT.at[:, pl.ds(e0, BE)], w_v)
                    for k in range(NSUB):
                        pltpu.sync_copy(xdt.at[idxa_v.at[k]],
                                        x1_v.at[pl.ds(k * SUB, SUB)])

                def grp_body(g, _):
                    gb = g * 16
                    if ps == 0:
                        wb = [w_v[k, pl.ds(gb, 16)] for k in range(3)]
                    else:
                        wb = [w_v[k, pl.ds(gb, 16)] for k in range(5)]
                    for l in range(16):
                        e = gb + l
                        lane = jnp.full((16,), l, jnp.int32)
                        if ps == 0:
                            msg_v[e, 0] = g1_v[e] * x1_v[e]
                            bp = g2_v[e] * x2_v[e]
                            for k in range(3):
                                w = wb[k].at[lane].get(
                                    mode="promise_in_bounds")
                                msg_v[e, 1 + k] = w * bp
                            msg_v[e, 4] = zv
                        else:
                            bd = g1_v[e] * x1_v[e]
                            for k in range(5):
                                w = wb[k].at[lane].get(
                                    mode="promise_in_bounds")
                                msg_v[e, k] = w * bd
                    return 0

                lax.fori_loop(0, BE // 16, grp_body, 0)
                for k in range(NSUB):
                    pltpu.sync_copy(msg_v.at[pl.ds(k * SUB, SUB)],
                                    acc.at[idxi_v.at[k]], add=True)
                return 0

            lax.fori_loop(0, NBLK, blk_body, 0)
            plsc.subcore_barrier()
            pltpu.sync_copy(acc.at[pl.ds(nb, RPT)],
                            out.at[cid, c, ps, pl.ds(nb, RPT)])


def _sc_scatter(gs, gp, gd, xst, xpt, xdt, pijT, dijT, idxi_r, idxj8):
    mesh = plsc.VectorSubcoreMesh(core_axis_name="c", subcore_axis_name="s",
                                  num_cores=NC, num_subcores=NS)
    fn = pl.kernel(
        _sc_body,
        out_type=jax.ShapeDtypeStruct((NC, NFC, 2, NPAD, PCH, FC),
                                      jnp.float32),
        mesh=mesh,
        scratch_types=[
            pltpu.VMEM((NSUB, SUB), jnp.int32),
            pltpu.VMEM((NSUB, SUB), jnp.int32),
            pltpu.VMEM((BE, FC), jnp.float32),
            pltpu.VMEM((BE, FC), jnp.float32),
            pltpu.VMEM((BE, FC), jnp.float32),
            pltpu.VMEM((BE, FC), jnp.float32),
            pltpu.VMEM((5, BE), jnp.float32),
            pltpu.VMEM((BE, PCH, FC), jnp.float32),
            pltpu.VMEM((ZROWS, PCH, FC), jnp.float32),
            pltpu.VMEM_SHARED((NPAD, PCH, FC), jnp.float32),
        ],
        compiler_params=pltpu.CompilerParams(use_tc_tiling_on_sc=False),
    )
    return fn(gs, gp, gd, xst, xpt, xdt, pijT, dijT, idxi_r, idxj8)


def _final_body(agg_ref, xx_ref, pp_ref, pd_ref, *wrefs):
    w = [r[...] for r in wrefs[:6]]
    out_ref = wrefs[6]
    a = agg_ref[0] + agg_ref[1]
    o = xx_ref[...] + a[:, 0, :]
    ppt = pp_ref[...]
    pdt = pd_ref[...]
    for k in range(1, 4):
        pr = jnp.dot(a[:, k, :], ppt, preferred_element_type=jnp.float32)
        o = o + pr[:, :F] * pr[:, F:]
    for k in range(4, 9):
        dr = jnp.dot(a[:, k, :], pdt, preferred_element_type=jnp.float32)
        o = o + dr[:, :F] * dr[:, F:]
    out_ref[...] = _resmlp(o, *w)


def _final(agg, xx, ppt, pdt, wflat):
    blk = 400
    grid = (N // blk,)
    wspecs = [pl.BlockSpec(a.shape, lambda i: (0,) * a.ndim)
              for a in [ppt, pdt] + list(wflat)]
    return pl.pallas_call(
        _final_body,
        grid=grid,
        in_specs=[pl.BlockSpec((NC, blk, NCH, F), lambda i: (0, i, 0, 0)),
                  pl.BlockSpec((blk, F), lambda i: (i, 0))] + wspecs,
        out_specs=pl.BlockSpec((blk, F), lambda i: (i, 0)),
        out_shape=jax.ShapeDtypeStruct((N, F), jnp.float32),
    )(agg, xx, ppt, pdt, *wflat)


def _mlp_weights(p):
    blk = p["blocks"][0]
    return [blk["lin1"]["w"].T, blk["lin1"]["b"].reshape(1, F),
            blk["lin2"]["w"].T, blk["lin2"]["b"].reshape(1, F),
            p["out"]["w"].T, p["out"]["b"].reshape(1, F)]


def kernel(x, rbf, pij, dij, idx_i, idx_j, params):
    wnode = []
    for name in ("resblock_x", "resblock_s", "resblock_p", "resblock_d"):
        wnode += _mlp_weights(params[name])
    xx, xsf, xpf, xdf = _node_mlps(x, wnode)

    def table(t):
        return t.reshape(N, NFC, FC).transpose(1, 0, 2).reshape(NFC * N, FC)

    xst, xpt, xdt = table(xsf), table(xpf), table(xdf)

    gs, gp, gd = _radial(rbf, params["radial_s"].T, params["radial_p"].T,
                         params["radial_d"].T)

    pijT = pij.T
    dijT = dij.T
    idxi_r = idx_i.reshape(P // SUB, SUB)
    idxj8 = (idx_j[None, :]
             + (jnp.arange(NFC, dtype=jnp.int32) * N)[:, None]).reshape(
                 NFC, P // SUB, SUB)

    out_sc = _sc_scatter(gs, gp, gd, xst, xpt, xdt, pijT, dijT, idxi_r, idxj8)

    agg9 = jnp.concatenate(
        [out_sc[:, :, 0, :, :NCH - PCH], out_sc[:, :, 1]], axis=3)
    agg = agg9.transpose(0, 2, 3, 1, 4).reshape(NC, NPAD, NCH, F)[:, :N]

    return _final(agg, xx, params["projection_p"].T, params["projection_d"].T,
                  _mlp_weights(params["resblock"]))


# speedup vs baseline: 10.7991x; 1.2598x over previous
import functools

import jax
import jax.numpy as jnp
from jax import lax
from jax.experimental import pallas as pl
from jax.experimental.pallas import tpu as pltpu
from jax.experimental.pallas import tpu_sc as plsc

N = 10000
P = 320000
F = 128
NBF = 32

NC = 2
NS = 16
NW = NC * NS
EPT = P // NW
BE = 400
NBLK = EPT // BE
SUB = 100
NSUB = BE // SUB
NCH = 9
FC = 16
NFC = F // FC
NPAD = 10240
RPT = NPAD // NS
ZROWS = 64
PCH = 5


def _swish(v):
    return v * jax.nn.sigmoid(v)


def _resmlp(v, w1t, b1, w2t, b2, wot, bot):
    y = _swish(v)
    y = jnp.dot(y, w1t, preferred_element_type=jnp.float32) + b1
    y = _swish(y)
    y = jnp.dot(y, w2t, preferred_element_type=jnp.float32) + b2
    v = v + y
    v = _swish(v)
    return jnp.dot(v, wot, preferred_element_type=jnp.float32) + bot


def _node_mlps_body(x_ref, *refs):
    w = [r[...] for r in refs[:24]]
    outs = refs[24:]
    xb = x_ref[...]
    for i in range(4):
        outs[i][...] = _resmlp(xb, *w[6 * i:6 * i + 6])


def _node_mlps(x, wflat):
    blk = 400
    grid = (N // blk,)
    wspecs = []
    for a in wflat:
        wspecs.append(pl.BlockSpec(a.shape, lambda i: (0,) * a.ndim))
    return pl.pallas_call(
        _node_mlps_body,
        grid=grid,
        in_specs=[pl.BlockSpec((blk, F), lambda i: (i, 0))] + wspecs,
        out_specs=[pl.BlockSpec((blk, F), lambda i: (i, 0))] * 4,
        out_shape=[jax.ShapeDtypeStruct((N, F), jnp.float32)] * 4,
    )(x, *wflat)


def _radial_body(rbf_ref, ws_ref, wp_ref, wd_ref, gs_ref, gp_ref, gd_ref):
    r = rbf_ref[...]
    gs_ref[...] = jnp.dot(r, ws_ref[...], preferred_element_type=jnp.float32)
    gp_ref[...] = jnp.dot(r, wp_ref[...], preferred_element_type=jnp.float32)
    gd_ref[...] = jnp.dot(r, wd_ref[...], preferred_element_type=jnp.float32)


def _radial(rbf, wst, wpt, wdt):
    blk = 3200
    grid = (P // blk,)
    wspec = pl.BlockSpec((NBF, F), lambda i: (0, 0))
    return pl.pallas_call(
        _radial_body,
        grid=grid,
        in_specs=[pl.BlockSpec((blk, NBF), lambda i: (i, 0)), wspec, wspec, wspec],
        out_specs=[pl.BlockSpec((blk, F), lambda i: (i, 0))] * 3,
        out_shape=[jax.ShapeDtypeStruct((P, F), jnp.float32)] * 3,
    )(rbf, wst, wpt, wdt)


def _sc_body(gs, gp, gd, xst, xpt, xdt, pijT, dijT, idxi_r, idxj8, out,
             idxi_v, idxa_v, g1_v, g2_v, x1_v, x2_v,
             w_v, msg_v, zbuf, acc, semM):
    cid = lax.axis_index("c")
    sid = lax.axis_index("s")
    wid = cid * NS + sid
    erow0 = wid * (EPT // SUB)
    nb = sid * RPT
    zv = jnp.zeros((FC,), jnp.float32)

    def zb_body(i, _):
        for ch in range(PCH):
            zbuf[i, ch] = zv
        return 0

    lax.fori_loop(0, ZROWS, zb_body, 0)

    def emit_idx(blk, c):
        r0 = erow0 + blk * NSUB
        pltpu.sync_copy(idxi_r.at[pl.ds(r0, NSUB)], idxi_v)
        pltpu.sync_copy(idxj8.at[c, pl.ds(r0, NSUB)], idxa_v)

    def emit_loads(blk, c, ps, issue):
        e0 = wid * EPT + blk * BE

        def cp(src, dst):
            if issue:
                pltpu.async_copy(src, dst, semM)
            else:
                pltpu.make_async_copy(src, dst, semM).wait()

        fsl = pl.ds(c * FC, FC)
        if ps == 0:
            cp(gs.at[pl.ds(e0, BE), fsl], g1_v)
            cp(gp.at[pl.ds(e0, BE), fsl], g2_v)
            cp(pijT.at[:, pl.ds(e0, BE)], w_v.at[pl.ds(0, 3)])
            for k in range(NSUB):
                cp(xst.at[idxa_v.at[k]], x1_v.at[pl.ds(k * SUB, SUB)])
                cp(xpt.at[idxa_v.at[k]], x2_v.at[pl.ds(k * SUB, SUB)])
        else:
            cp(gd.at[pl.ds(e0, BE), fsl], g1_v)
            cp(dijT.at[:, pl.ds(e0, BE)], w_v)
            for k in range(NSUB):
                cp(xdt.at[idxa_v.at[k]], x1_v.at[pl.ds(k * SUB, SUB)])

    def emit_scatter():
        for k in range(NSUB):
            pltpu.sync_copy(msg_v.at[pl.ds(k * SUB, SUB)],
                            acc.at[idxi_v.at[k]], add=True)

    def compute(ps):
        def grp_body(g, _):
            gb = g * 16
            nw = 3 if ps == 0 else 5
            wb = [w_v[k, pl.ds(gb, 16)] for k in range(nw)]
            for l in range(16):
                e = gb + l
                lane = jnp.full((16,), l, jnp.int32)
                if ps == 0:
                    msg_v[e, 0] = g1_v[e] * x1_v[e]
                    bp = g2_v[e] * x2_v[e]
                    for k in range(3):
                        w = wb[k].at[lane].get(mode="promise_in_bounds")
                        msg_v[e, 1 + k] = w * bp
                    msg_v[e, 4] = zv
                else:
                    bd = g1_v[e] * x1_v[e]
                    for k in range(5):
                        w = wb[k].at[lane].get(mode="promise_in_bounds")
                        msg_v[e, k] = w * bd
            return 0

        lax.fori_loop(0, BE // 16, grp_body, 0)

    for c in range(NFC):
        for ps in range(2):
            for z in range(RPT // ZROWS):
                pltpu.sync_copy(zbuf, acc.at[pl.ds(nb + z * ZROWS, ZROWS)])
            plsc.subcore_barrier()

            def blk_body(blk, _):
                emit_idx(blk, c)
                emit_loads(blk, c, ps, True)
                emit_loads(blk, c, ps, False)
                compute(ps)
                emit_scatter()
                return 0

            lax.fori_loop(0, NBLK, blk_body, 0)
            plsc.subcore_barrier()
            pltpu.sync_copy(acc.at[pl.ds(nb, RPT)],
                            out.at[cid, c, ps, pl.ds(nb, RPT)])


def _sc_scatter(gs, gp, gd, xst, xpt, xdt, pijT, dijT, idxi_r, idxj8):
    mesh = plsc.VectorSubcoreMesh(core_axis_name="c", subcore_axis_name="s",
                                  num_cores=NC, num_subcores=NS)
    fn = pl.kernel(
        _sc_body,
        out_type=jax.ShapeDtypeStruct((NC, NFC, 2, NPAD, PCH, FC),
                                      jnp.float32),
        mesh=mesh,
        scratch_types=[
            pltpu.VMEM((NSUB, SUB), jnp.int32),
            pltpu.VMEM((NSUB, SUB), jnp.int32),
            pltpu.VMEM((BE, FC), jnp.float32),
            pltpu.VMEM((BE, FC), jnp.float32),
            pltpu.VMEM((BE, FC), jnp.float32),
            pltpu.VMEM((BE, FC), jnp.float32),
            pltpu.VMEM((5, BE), jnp.float32),
            pltpu.VMEM((BE, PCH, FC), jnp.float32),
            pltpu.VMEM((ZROWS, PCH, FC), jnp.float32),
            pltpu.VMEM_SHARED((NPAD, PCH, FC), jnp.float32),
            pltpu.SemaphoreType.DMA,
        ],
        compiler_params=pltpu.CompilerParams(use_tc_tiling_on_sc=False),
    )
    return fn(gs, gp, gd, xst, xpt, xdt, pijT, dijT, idxi_r, idxj8)


def _final_body(agg_ref, xx_ref, pp_ref, pd_ref, *wrefs):
    w = [r[...] for r in wrefs[:6]]
    out_ref = wrefs[6]
    a = agg_ref[0] + agg_ref[1]
    o = xx_ref[...] + a[:, 0, :]
    ppt = pp_ref[...]
    pdt = pd_ref[...]
    for k in range(1, 4):
        pr = jnp.dot(a[:, k, :], ppt, preferred_element_type=jnp.float32)
        o = o + pr[:, :F] * pr[:, F:]
    for k in range(4, 9):
        dr = jnp.dot(a[:, k, :], pdt, preferred_element_type=jnp.float32)
        o = o + dr[:, :F] * dr[:, F:]
    out_ref[...] = _resmlp(o, *w)


def _final(agg, xx, ppt, pdt, wflat):
    blk = 400
    grid = (N // blk,)
    wspecs = [pl.BlockSpec(a.shape, lambda i: (0,) * a.ndim)
              for a in [ppt, pdt] + list(wflat)]
    return pl.pallas_call(
        _final_body,
        grid=grid,
        in_specs=[pl.BlockSpec((NC, blk, NCH, F), lambda i: (0, i, 0, 0)),
                  pl.BlockSpec((blk, F), lambda i: (i, 0))] + wspecs,
        out_specs=pl.BlockSpec((blk, F), lambda i: (i, 0)),
        out_shape=jax.ShapeDtypeStruct((N, F), jnp.float32),
    )(agg, xx, ppt, pdt, *wflat)


def _mlp_weights(p):
    blk = p["blocks"][0]
    return [blk["lin1"]["w"].T, blk["lin1"]["b"].reshape(1, F),
            blk["lin2"]["w"].T, blk["lin2"]["b"].reshape(1, F),
            p["out"]["w"].T, p["out"]["b"].reshape(1, F)]


def kernel(x, rbf, pij, dij, idx_i, idx_j, params):
    wnode = []
    for name in ("resblock_x", "resblock_s", "resblock_p", "resblock_d"):
        wnode += _mlp_weights(params[name])
    xx, xsf, xpf, xdf = _node_mlps(x, wnode)

    def table(t):
        return t.reshape(N, NFC, FC).transpose(1, 0, 2).reshape(NFC * N, FC)

    xst, xpt, xdt = table(xsf), table(xpf), table(xdf)

    gs, gp, gd = _radial(rbf, params["radial_s"].T, params["radial_p"].T,
                         params["radial_d"].T)

    pijT = pij.T
    dijT = dij.T
    idxi_r = idx_i.reshape(P // SUB, SUB)
    idxj8 = (idx_j[None, :]
             + (jnp.arange(NFC, dtype=jnp.int32) * N)[:, None]).reshape(
                 NFC, P // SUB, SUB)

    out_sc = _sc_scatter(gs, gp, gd, xst, xpt, xdt, pijT, dijT, idxi_r, idxj8)

    agg9 = jnp.concatenate(
        [out_sc[:, :, 0, :, :NCH - PCH], out_sc[:, :, 1]], axis=3)
    agg = agg9.transpose(0, 2, 3, 1, 4).reshape(NC, NPAD, NCH, F)[:, :N]

    return _final(agg, xx, params["projection_p"].T, params["projection_d"].T,
                  _mlp_weights(params["resblock"]))


# speedup vs baseline: 11.4454x; 1.0598x over previous
import functools

import jax
import jax.numpy as jnp
from jax import lax
from jax.experimental import pallas as pl
from jax.experimental.pallas import tpu as pltpu
from jax.experimental.pallas import tpu_sc as plsc

N = 10000
P = 320000
F = 128
NBF = 32

NC = 2
NS = 16
NW = NC * NS
EPT = P // NW
BE = 400
NBLK = EPT // BE
SUB = 100
NSUB = BE // SUB
NCH = 9
FC = 16
NFC = F // FC
NPAD = 10240
RPT = NPAD // NS
ZROWS = 64
PCH = 5


def _swish(v):
    return v * jax.nn.sigmoid(v)


def _resmlp(v, w1t, b1, w2t, b2, wot, bot):
    y = _swish(v)
    y = jnp.dot(y, w1t, preferred_element_type=jnp.float32) + b1
    y = _swish(y)
    y = jnp.dot(y, w2t, preferred_element_type=jnp.float32) + b2
    v = v + y
    v = _swish(v)
    return jnp.dot(v, wot, preferred_element_type=jnp.float32) + bot


def _node_mlps_body(x_ref, *refs):
    w = [r[...] for r in refs[:24]]
    outs = refs[24:]
    xb = x_ref[...]
    for i in range(4):
        outs[i][...] = _resmlp(xb, *w[6 * i:6 * i + 6])


def _node_mlps(x, wflat):
    blk = 400
    grid = (N // blk,)
    wspecs = []
    for a in wflat:
        wspecs.append(pl.BlockSpec(a.shape, lambda i: (0,) * a.ndim))
    return pl.pallas_call(
        _node_mlps_body,
        grid=grid,
        in_specs=[pl.BlockSpec((blk, F), lambda i: (i, 0))] + wspecs,
        out_specs=[pl.BlockSpec((blk, F), lambda i: (i, 0))] * 4,
        out_shape=[jax.ShapeDtypeStruct((N, F), jnp.float32)] * 4,
    )(x, *wflat)


def _radial_body(rbf_ref, ws_ref, wp_ref, wd_ref, gs_ref, gp_ref, gd_ref):
    r = rbf_ref[...]
    gs_ref[...] = jnp.dot(r, ws_ref[...], preferred_element_type=jnp.float32)
    gp_ref[...] = jnp.dot(r, wp_ref[...], preferred_element_type=jnp.float32)
    gd_ref[...] = jnp.dot(r, wd_ref[...], preferred_element_type=jnp.float32)


def _radial(rbf, wst, wpt, wdt):
    blk = 3200
    grid = (P // blk,)
    wspec = pl.BlockSpec((NBF, F), lambda i: (0, 0))
    return pl.pallas_call(
        _radial_body,
        grid=grid,
        in_specs=[pl.BlockSpec((blk, NBF), lambda i: (i, 0)), wspec, wspec, wspec],
        out_specs=[pl.BlockSpec((blk, F), lambda i: (i, 0))] * 3,
        out_shape=[jax.ShapeDtypeStruct((P, F), jnp.float32)] * 3,
    )(rbf, wst, wpt, wdt)


def _sc_body(gs, gp, gd, xspt, xdt, pijT, dijT, idxc, out,
             idxc_v, g1_v, g2_v, x12_v, x1_v,
             w_v, msg_v, zbuf, acc, semM, semS):
    cid = lax.axis_index("c")
    sid = lax.axis_index("s")
    wid = cid * NS + sid
    erow0 = wid * (EPT // SUB) * 2
    nb = sid * RPT
    zv = jnp.zeros((FC,), jnp.float32)

    def zb_body(i, _):
        for ch in range(PCH):
            zbuf[i, ch] = zv
        return 0

    lax.fori_loop(0, ZROWS, zb_body, 0)

    def emit_loads(blk, c, ps, issue):
        e0 = wid * EPT + blk * BE

        def cp(src, dst):
            if issue:
                pltpu.async_copy(src, dst, semM)
            else:
                pltpu.make_async_copy(src, dst, semM).wait()

        fsl = pl.ds(c * FC, FC)
        if ps == 0:
            cp(gs.at[pl.ds(e0, BE), fsl], g1_v)
            cp(gp.at[pl.ds(e0, BE), fsl], g2_v)
            cp(pijT.at[:, pl.ds(e0, BE)], w_v.at[pl.ds(0, 3)])
            for k in range(NSUB):
                cp(xspt.at[idxc_v.at[2 * k]],
                   x12_v.at[pl.ds(k * SUB, SUB)])
        else:
            cp(gd.at[pl.ds(e0, BE), fsl], g1_v)
            cp(dijT.at[:, pl.ds(e0, BE)], w_v)
            for k in range(NSUB):
                cp(xdt.at[idxc_v.at[2 * k]],
                   x1_v.at[pl.ds(k * SUB, SUB)])

    def emit_scatter(issue):
        for k in range(NSUB):
            src = msg_v.at[pl.ds(k * SUB, SUB)]
            dst = acc.at[idxc_v.at[2 * k + 1]]
            if issue:
                pltpu.async_copy(src, dst, semS, add=True)
            else:
                pltpu.make_async_copy(src, dst, semS).wait()

    def compute(ps):
        def grp_body(g, _):
            gb = g * 16
            nw = 3 if ps == 0 else 5
            wb = [w_v[k, pl.ds(gb, 16)] for k in range(nw)]
            for l in range(16):
                e = gb + l
                lane = jnp.full((16,), l, jnp.int32)
                if ps == 0:
                    msg_v[e, 0] = g1_v[e] * x12_v[e, pl.ds(0, FC)]
                    bp = g2_v[e] * x12_v[e, pl.ds(FC, FC)]
                    for k in range(3):
                        w = wb[k].at[lane].get(mode="promise_in_bounds")
                        msg_v[e, 1 + k] = w * bp
                    msg_v[e, 4] = zv
                else:
                    bd = g1_v[e] * x1_v[e]
                    for k in range(5):
                        w = wb[k].at[lane].get(mode="promise_in_bounds")
                        msg_v[e, k] = w * bd
            return 0

        lax.fori_loop(0, BE // 16, grp_body, 0)

    for c in range(NFC):
        for ps in range(2):
            for z in range(RPT // ZROWS):
                pltpu.async_copy(zbuf,
                                 acc.at[pl.ds(nb + z * ZROWS, ZROWS)], semM)
            for z in range(RPT // ZROWS):
                pltpu.make_async_copy(
                    zbuf, acc.at[pl.ds(nb + z * ZROWS, ZROWS)], semM).wait()
            plsc.subcore_barrier()

            def blk_body(blk, _):
                r0 = erow0 + blk * NSUB * 2
                pltpu.sync_copy(idxc.at[c, pl.ds(r0, 2 * NSUB)], idxc_v)
                emit_loads(blk, c, ps, True)
                emit_loads(blk, c, ps, False)
                compute(ps)
                emit_scatter(True)
                emit_scatter(False)
                return 0

            lax.fori_loop(0, NBLK, blk_body, 0)
            plsc.subcore_barrier()
            pltpu.sync_copy(acc.at[pl.ds(nb, RPT)],
                            out.at[cid, c, ps, pl.ds(nb, RPT)])


def _sc_scatter(gs, gp, gd, xspt, xdt, pijT, dijT, idxc):
    mesh = plsc.VectorSubcoreMesh(core_axis_name="c", subcore_axis_name="s",
                                  num_cores=NC, num_subcores=NS)
    fn = pl.kernel(
        _sc_body,
        out_type=jax.ShapeDtypeStruct((NC, NFC, 2, NPAD, PCH, FC),
                                      jnp.float32),
        mesh=mesh,
        scratch_types=[
            pltpu.VMEM((2 * NSUB, SUB), jnp.int32),
            pltpu.VMEM((BE, FC), jnp.float32),
            pltpu.VMEM((BE, FC), jnp.float32),
            pltpu.VMEM((BE, 2 * FC), jnp.float32),
            pltpu.VMEM((BE, FC), jnp.float32),
            pltpu.VMEM((5, BE), jnp.float32),
            pltpu.VMEM((BE, PCH, FC), jnp.float32),
            pltpu.VMEM((ZROWS, PCH, FC), jnp.float32),
            pltpu.VMEM_SHARED((NPAD, PCH, FC), jnp.float32),
            pltpu.SemaphoreType.DMA,
            pltpu.SemaphoreType.DMA,
        ],
        compiler_params=pltpu.CompilerParams(use_tc_tiling_on_sc=False),
    )
    return fn(gs, gp, gd, xspt, xdt, pijT, dijT, idxc)


def _final_body(agg_ref, xx_ref, pp_ref, pd_ref, *wrefs):
    w = [r[...] for r in wrefs[:6]]
    out_ref = wrefs[6]
    a = agg_ref[0] + agg_ref[1]
    o = xx_ref[...] + a[:, 0, :]
    ppt = pp_ref[...]
    pdt = pd_ref[...]
    for k in range(1, 4):
        pr = jnp.dot(a[:, k, :], ppt, preferred_element_type=jnp.float32)
        o = o + pr[:, :F] * pr[:, F:]
    for k in range(4, 9):
        dr = jnp.dot(a[:, k, :], pdt, preferred_element_type=jnp.float32)
        o = o + dr[:, :F] * dr[:, F:]
    out_ref[...] = _resmlp(o, *w)


def _final(agg, xx, ppt, pdt, wflat):
    blk = 400
    grid = (N // blk,)
    wspecs = [pl.BlockSpec(a.shape, lambda i: (0,) * a.ndim)
              for a in [ppt, pdt] + list(wflat)]
    return pl.pallas_call(
        _final_body,
        grid=grid,
        in_specs=[pl.BlockSpec((NC, blk, NCH, F), lambda i: (0, i, 0, 0)),
                  pl.BlockSpec((blk, F), lambda i: (i, 0))] + wspecs,
        out_specs=pl.BlockSpec((blk, F), lambda i: (i, 0)),
        out_shape=jax.ShapeDtypeStruct((N, F), jnp.float32),
    )(agg, xx, ppt, pdt, *wflat)


def _mlp_weights(p):
    blk = p["blocks"][0]
    return [blk["lin1"]["w"].T, blk["lin1"]["b"].reshape(1, F),
            blk["lin2"]["w"].T, blk["lin2"]["b"].reshape(1, F),
            p["out"]["w"].T, p["out"]["b"].reshape(1, F)]


def kernel(x, rbf, pij, dij, idx_i, idx_j, params):
    wnode = []
    for name in ("resblock_x", "resblock_s", "resblock_p", "resblock_d"):
        wnode += _mlp_weights(params[name])
    xx, xsf, xpf, xdf = _node_mlps(x, wnode)

    def table(t):
        return t.reshape(N, NFC, FC).transpose(1, 0, 2).reshape(NFC * N, FC)

    xspt = jnp.concatenate(
        [xsf.reshape(N, NFC, 1, FC), xpf.reshape(N, NFC, 1, FC)],
        axis=2).transpose(1, 0, 2, 3).reshape(NFC * N, 2 * FC)
    xdt = table(xdf)

    gs, gp, gd = _radial(rbf, params["radial_s"].T, params["radial_p"].T,
                         params["radial_d"].T)

    pijT = pij.T
    dijT = dij.T
    idxj8 = (idx_j[None, :]
             + (jnp.arange(NFC, dtype=jnp.int32) * N)[:, None]).reshape(
                 NFC, P // SUB, SUB)
    idxir = jnp.broadcast_to(idx_i.reshape(1, P // SUB, SUB),
                             (NFC, P // SUB, SUB))
    idxc = jnp.stack([idxj8, idxir], axis=2).reshape(
        NFC, 2 * (P // SUB), SUB)

    out_sc = _sc_scatter(gs, gp, gd, xspt, xdt, pijT, dijT, idxc)

    agg9 = jnp.concatenate(
        [out_sc[:, :, 0, :, :NCH - PCH], out_sc[:, :, 1]], axis=3)
    agg = agg9.transpose(0, 2, 3, 1, 4).reshape(NC, NPAD, NCH, F)

    return _final(agg, xx, params["projection_p"].T, params["projection_d"].T,
                  _mlp_weights(params["resblock"]))
